# baseline (speedup 1.0000x reference)
import functools
import math

import jax
import jax.numpy as jnp
from jax import lax
from jax.experimental import pallas as pl
from jax.experimental.pallas import tpu as pltpu
from jax.experimental.pallas import tpu_sc as plsc

N_NODES = 10000
HIDDEN = 128
NUM_FILTERS = 128
NUM_GAUSSIANS = 50
N_EDGES = 320000
CUTOFF = 10.0
SHIFT = float(math.log(2.0))

NC, NS = 2, 16
NW = NC * NS
EPW = N_EDGES // NW
CH = 40
NCH = EPW // CH
NBUF = 3
NSL = 6
NGRP = 41
SLAB_OFF = 624
SLAB = 640


def _ssp(v):
    return jnp.maximum(v, 0.0) + jnp.log1p(jnp.exp(-jnp.abs(v))) - SHIFT


def _xf_body(x_ref, w_ref, o_ref):
    o_ref[...] = jnp.dot(x_ref[...], w_ref[...],
                         preferred_element_type=jnp.float32)


def _compute_xf(x, lin1_w):
    BR = 2000
    return pl.pallas_call(
        _xf_body,
        grid=(N_NODES // BR,),
        in_specs=[pl.BlockSpec((BR, HIDDEN), lambda i: (i, 0)),
                  pl.BlockSpec((HIDDEN, NUM_FILTERS), lambda i: (0, 0))],
        out_specs=pl.BlockSpec((BR, NUM_FILTERS), lambda i: (i, 0)),
        out_shape=jax.ShapeDtypeStruct((N_NODES, NUM_FILTERS), jnp.float32),
    )(x, lin1_w)


def _w_body(ea_ref, ew_ref, w1_ref, b1_ref, w2_ref, b2_ref, o_ref):
    h = jnp.dot(ea_ref[...], w1_ref[...],
                preferred_element_type=jnp.float32) + b1_ref[...]
    h = _ssp(h)
    w = jnp.dot(h, w2_ref[...],
                preferred_element_type=jnp.float32) + b2_ref[...]
    cfac = 0.5 * (jnp.cos(ew_ref[...] * (math.pi / CUTOFF)) + 1.0)
    o_ref[...] = w * cfac


def _compute_w(edge_attr, ew2, w1, b1, w2, b2):
    BE = 2000
    return pl.pallas_call(
        _w_body,
        grid=(N_EDGES // BE,),
        in_specs=[pl.BlockSpec((BE, NUM_GAUSSIANS), lambda i: (i, 0)),
                  pl.BlockSpec((BE, 1), lambda i: (i, 0)),
                  pl.BlockSpec((NUM_GAUSSIANS, NUM_FILTERS), lambda i: (0, 0)),
                  pl.BlockSpec((1, NUM_FILTERS), lambda i: (0, 0)),
                  pl.BlockSpec((NUM_FILTERS, NUM_FILTERS), lambda i: (0, 0)),
                  pl.BlockSpec((1, NUM_FILTERS), lambda i: (0, 0))],
        out_specs=pl.BlockSpec((BE, NUM_FILTERS), lambda i: (i, 0)),
        out_shape=jax.ShapeDtypeStruct((N_EDGES, NUM_FILTERS), jnp.float32),
    )(edge_attr, ew2, w1, b1, w2, b2)


def _tail_body(p_ref, w2_ref, b2_ref, lw_ref, lb_ref, o_ref):
    agg = p_ref[0] + p_ref[1]
    t = jnp.dot(agg, w2_ref[...],
                preferred_element_type=jnp.float32) + b2_ref[...]
    t = _ssp(t)
    o_ref[...] = jnp.dot(t, lw_ref[...],
                         preferred_element_type=jnp.float32) + lb_ref[...]


def _compute_tail(parts, lin2_w, lin2_b, lin_w, lin_b):
    BR = 2000
    return pl.pallas_call(
        _tail_body,
        grid=(N_NODES // BR,),
        in_specs=[pl.BlockSpec((2, BR, NUM_FILTERS), lambda i: (0, i, 0)),
                  pl.BlockSpec((NUM_FILTERS, HIDDEN), lambda i: (0, 0)),
                  pl.BlockSpec((1, HIDDEN), lambda i: (0, 0)),
                  pl.BlockSpec((HIDDEN, HIDDEN), lambda i: (0, 0)),
                  pl.BlockSpec((1, HIDDEN), lambda i: (0, 0))],
        out_specs=pl.BlockSpec((BR, HIDDEN), lambda i: (i, 0)),
        out_shape=jax.ShapeDtypeStruct((N_NODES, HIDDEN), jnp.float32),
    )(parts, lin2_w, lin2_b, lin_w, lin_b)


_sc_mesh = plsc.VectorSubcoreMesh(core_axis_name="c", subcore_axis_name="s")


@functools.partial(
    pl.kernel, mesh=_sc_mesh,
    out_type=jax.ShapeDtypeStruct((NC, N_NODES, NUM_FILTERS), jnp.float32),
    scratch_types=[
        pltpu.VMEM((NSL, 2, CH), jnp.int32),
        pltpu.VMEM((NBUF, CH, NUM_FILTERS), jnp.float32),
        pltpu.VMEM((NBUF, CH, NUM_FILTERS), jnp.float32),
        pltpu.SemaphoreType.DMA((NSL,)),
        pltpu.SemaphoreType.DMA((NBUF,)),
        pltpu.SemaphoreType.DMA((NBUF,)),
        pltpu.SemaphoreType.DMA((NBUF,)),
        pltpu.VMEM_SHARED((N_NODES, NUM_FILTERS), jnp.float32),
    ])
def _sc_fused(xf_hbm, wmat_hbm, idx_hbm, zero_hbm, out_hbm,
              idx_v, g_v, w_v, isem, gsem, wsem, dsem, acc_sh):
    c = lax.axis_index("c")
    s = lax.axis_index("s")
    wid = s * NC + c
    base = wid * EPW

    pltpu.sync_copy(zero_hbm.at[pl.ds(s * SLAB_OFF, SLAB)],
                    acc_sh.at[pl.ds(s * SLAB_OFF, SLAB)])

    def issue_idx(k, sl):
        pltpu.make_async_copy(idx_hbm.at[wid, k], idx_v.at[sl],
                              isem.at[sl]).start()

    def wait_idx(sl):
        pltpu.make_async_copy(idx_hbm.at[wid, 0], idx_v.at[sl],
                              isem.at[sl]).wait()

    def issue_in(k, j, sl):
        pltpu.make_async_copy(xf_hbm.at[idx_v.at[sl, 0]], g_v.at[j],
                              gsem.at[j]).start()
        pltpu.make_async_copy(wmat_hbm.at[pl.ds(base + k * CH, CH)],
                              w_v.at[j], wsem.at[j]).start()

    def wait_in(j):
        pltpu.make_async_copy(xf_hbm.at[idx_v.at[0, 0]], g_v.at[j],
                              gsem.at[j]).wait()
        pltpu.make_async_copy(wmat_hbm.at[pl.ds(0, CH)], w_v.at[j],
                              wsem.at[j]).wait()

    def mul_scatter(j, sl):
        gb = g_v.at[j]
        wb = w_v.at[j]

        @plsc.parallel_loop(0, CH, 1, unroll=4)
        def _(r):
            for l in range(8):
                s16 = pl.ds(l * 16, 16)
                gb[r, s16] = gb[r, s16] * wb[r, s16]

        pltpu.make_async_copy(gb, acc_sh.at[idx_v.at[sl, 1]],
                              dsem.at[j]).start(add=True)

    def wait_scatter(j):
        pltpu.make_async_copy(g_v.at[0], acc_sh.at[idx_v.at[0, 1]],
                              dsem.at[j]).wait()

    issue_idx(0, 0)
    issue_idx(1, 1)
    wait_idx(0)
    issue_in(0, 0, 0)
    plsc.subcore_barrier()

    def group(g, carry):
        k0 = g * NSL
        for j6 in range(NSL):
            k = k0 + j6
            j3 = j6 % NBUF
            wait_in(j3)
            mul_scatter(j3, j6)
            issue_idx(k + 2, (j6 + 2) % NSL)
            jn = (j6 + 1) % NBUF
            sn = (j6 + 1) % NSL
            if j6 >= 2:
                wait_scatter(jn)
            else:
                @pl.when(g > 0)
                def _():
                    wait_scatter(jn)
            wait_idx(sn)
            issue_in(k + 1, jn, sn)
        return carry

    lax.fori_loop(0, NGRP, group, 0)

    wait_in(0)
    mul_scatter(0, 0)
    issue_idx(NCH - 2, 2)
    wait_scatter(1)
    wait_idx(1)
    issue_in(NCH - 3, 1, 1)

    wait_in(1)
    mul_scatter(1, 1)
    issue_idx(NCH - 1, 3)
    wait_scatter(2)
    wait_idx(2)
    issue_in(NCH - 2, 2, 2)

    wait_in(2)
    mul_scatter(2, 2)
    wait_scatter(0)
    wait_idx(3)
    issue_in(NCH - 1, 0, 3)

    wait_in(0)
    mul_scatter(0, 3)

    wait_scatter(1)
    wait_scatter(2)
    wait_scatter(0)

    plsc.subcore_barrier()
    pltpu.sync_copy(acc_sh.at[pl.ds(s * SLAB_OFF, SLAB)],
                    out_hbm.at[c, pl.ds(s * SLAB_OFF, SLAB)])


def kernel(x, edge_index, edge_weight, edge_attr,
           lin1_w, lin2_w, lin2_b, mlp_w1, mlp_b1, mlp_w2, mlp_b2,
           lin_w, lin_b):
    ei = edge_index.astype(jnp.int32)
    rowr = ei[0].reshape(NW, NCH, CH)
    colr = ei[1].reshape(NW, NCH, CH)
    idx2 = jnp.stack([colr, rowr], axis=2)

    xf = _compute_xf(x, lin1_w)
    wmat = _compute_w(edge_attr, edge_weight.reshape(N_EDGES, 1),
                      mlp_w1, mlp_b1.reshape(1, NUM_FILTERS),
                      mlp_w2, mlp_b2.reshape(1, NUM_FILTERS))
    zeros = jnp.zeros((N_NODES, NUM_FILTERS), jnp.float32)
    parts = _sc_fused(xf, wmat, idx2, zeros)
    out = _compute_tail(parts, lin2_w, lin2_b.reshape(1, HIDDEN),
                        lin_w, lin_b.reshape(1, HIDDEN))
    return out
